# baseline (speedup 1.0000x reference)
import functools

import jax
import jax.numpy as jnp
from jax import lax
from jax.experimental import pallas as pl
from jax.experimental.pallas import tpu as pltpu
from jax.experimental.pallas import tpu_sc as plsc

N = 10000
E = 320000
NC, NS = 2, 16
NW = NC * NS
K = 80
NBD = E // NW // K
NBS = E // NS // K
NBUF = {64: 5, 32: 10}
TPB = N // NS

_SC_PARAMS = dict(compiler_params=pltpu.CompilerParams(use_tc_tiling_on_sc=False))


def _mesh():
    return plsc.VectorSubcoreMesh(core_axis_name="c", subcore_axis_name="s")


@functools.cache
def _get_sc_degree():
    @functools.partial(
        pl.kernel,
        mesh=_mesh(),
        out_type=jax.ShapeDtypeStruct((NC, N, 16), jnp.float32),
        scratch_types=[
            pltpu.VMEM((NBD, K), jnp.int32),
            pltpu.VMEM((K, 16), jnp.float32),
            pltpu.VMEM_SHARED((N, 16), jnp.float32),
        ],
        **_SC_PARAMS,
    )
    def _sc_degree(dst_hbm, ones_hbm, zeros_hbm, out_hbm, dstv, onesv, acc):
        c = lax.axis_index("c")
        s = lax.axis_index("s")
        wid = s * NC + c
        pltpu.sync_copy(zeros_hbm.at[pl.ds(s * TPB, TPB)], acc.at[pl.ds(s * TPB, TPB)])
        pltpu.sync_copy(dst_hbm.at[wid], dstv)
        pltpu.sync_copy(ones_hbm, onesv)
        plsc.subcore_barrier()

        def body(i, carry):
            pltpu.sync_copy(onesv, acc.at[dstv.at[i]], add=True)
            return carry

        lax.fori_loop(0, NBD, body, 0)
        plsc.subcore_barrier()
        pltpu.sync_copy(acc.at[pl.ds(s * TPB, TPB)], out_hbm.at[c, pl.ds(s * TPB, TPB)])

    return _sc_degree


@functools.cache
def _make_sc_segsum(Dh):
    nbuf = NBUF[Dh]

    @functools.partial(
        pl.kernel,
        mesh=_mesh(),
        out_type=jax.ShapeDtypeStruct((NC, N, Dh), jnp.float32),
        scratch_types=[
            pltpu.VMEM((NBS, K), jnp.int32),
            pltpu.VMEM((NBS, K), jnp.int32),
            [pltpu.VMEM((K, Dh), jnp.float32) for _ in range(nbuf)],
            pltpu.VMEM_SHARED((N, Dh), jnp.float32),
            pltpu.SemaphoreType.DMA,
        ],
        **_SC_PARAMS,
    )
    def _sc_segsum(y_hbm, src_hbm, dst_hbm, zeros_hbm, out_hbm,
                   srcv, dstv, bufs, acc, sem):
        c = lax.axis_index("c")
        s = lax.axis_index("s")
        pltpu.sync_copy(zeros_hbm.at[pl.ds(s * TPB, TPB)], acc.at[pl.ds(s * TPB, TPB)])
        pltpu.sync_copy(src_hbm.at[s], srcv)
        pltpu.sync_copy(dst_hbm.at[s], dstv)
        plsc.subcore_barrier()

        def run(cc):
            tbl = y_hbm.at[cc]
            for b in range(nbuf):
                pltpu.async_copy(tbl.at[srcv.at[b]], bufs[b], sem)

            def body(j, carry):
                for b in range(nbuf):
                    i = j * nbuf + b
                    pltpu.make_async_copy(tbl.at[srcv.at[i]], bufs[b], sem).wait()
                    pltpu.sync_copy(bufs[b], acc.at[dstv.at[i]], add=True)

                    @pl.when(i + nbuf < NBS)
                    def _():
                        pltpu.async_copy(tbl.at[srcv.at[i + nbuf]], bufs[b], sem)

                return carry

            lax.fori_loop(0, NBS // nbuf, body, 0)

        @pl.when(c == 0)
        def _():
            run(0)

        @pl.when(c == 1)
        def _():
            run(1)

        plsc.subcore_barrier()
        pltpu.sync_copy(acc.at[pl.ds(s * TPB, TPB)],
                        out_hbm.at[c, pl.ds(s * TPB, TPB)])

    return _sc_segsum


_BR = 1000


def _dinv_block(d0_ref, d1_ref):
    deg = 1.0 + d0_ref[:, 0:1] + d1_ref[:, 0:1]
    return lax.rsqrt(deg)


def _tc_matmul(x, W):
    F, H = W.shape

    def body(x_ref, w_ref, o_ref):
        o_ref[...] = jnp.dot(x_ref[...], w_ref[...],
                             preferred_element_type=jnp.float32)

    return pl.pallas_call(
        body,
        grid=(N // _BR,),
        in_specs=[
            pl.BlockSpec((_BR, F), lambda i: (i, 0)),
            pl.BlockSpec((F, H), lambda i: (0, 0)),
        ],
        out_specs=pl.BlockSpec((_BR, H), lambda i: (i, 0)),
        out_shape=jax.ShapeDtypeStruct((N, H), jnp.float32),
    )(x, W)


def _tc_scale_split(xw, d0, d1):
    H = xw.shape[1]

    def body(xw_ref, d0_ref, d1_ref, y_ref):
        dinv = _dinv_block(d0_ref, d1_ref)
        yblk = xw_ref[...] * dinv
        y_ref[0, :, :] = yblk[:, : H // 2]
        y_ref[1, :, :] = yblk[:, H // 2:]

    return pl.pallas_call(
        body,
        grid=(N // _BR,),
        in_specs=[
            pl.BlockSpec((_BR, H), lambda i: (i, 0)),
            pl.BlockSpec((_BR, 16), lambda i: (i, 0)),
            pl.BlockSpec((_BR, 16), lambda i: (i, 0)),
        ],
        out_specs=pl.BlockSpec((2, _BR, H // 2), lambda i: (0, i, 0)),
        out_shape=jax.ShapeDtypeStruct((2, N, H // 2), jnp.float32),
    )(xw, d0, d1)


def _tc_mid(a, y1, d0, d1, b1, W2):
    H, C = W2.shape

    def body(a_ref, y1_ref, d0_ref, d1_ref, b1_ref, w2_ref, y2_ref):
        dinv = _dinv_block(d0_ref, d1_ref)
        ssum = jnp.concatenate(
            [a_ref[0] + y1_ref[0], a_ref[1] + y1_ref[1]], axis=-1)
        h = jnp.maximum(dinv * ssum + b1_ref[...], 0.0)
        yy = jnp.dot(h, w2_ref[...], preferred_element_type=jnp.float32) * dinv
        y2_ref[0, :, :] = yy[:, : C // 2]
        y2_ref[1, :, :] = yy[:, C // 2:]

    return pl.pallas_call(
        body,
        grid=(N // _BR,),
        in_specs=[
            pl.BlockSpec((2, _BR, H // 2), lambda i: (0, i, 0)),
            pl.BlockSpec((2, _BR, H // 2), lambda i: (0, i, 0)),
            pl.BlockSpec((_BR, 16), lambda i: (i, 0)),
            pl.BlockSpec((_BR, 16), lambda i: (i, 0)),
            pl.BlockSpec((1, H), lambda i: (0, 0)),
            pl.BlockSpec((H, C), lambda i: (0, 0)),
        ],
        out_specs=pl.BlockSpec((2, _BR, C // 2), lambda i: (0, i, 0)),
        out_shape=jax.ShapeDtypeStruct((2, N, C // 2), jnp.float32),
    )(a, y1, d0, d1, b1, W2)


def _tc_final(a, y2, d0, d1, b2):
    C = 2 * y2.shape[2]

    def body(a_ref, y2_ref, d0_ref, d1_ref, b2_ref, o_ref):
        dinv = _dinv_block(d0_ref, d1_ref)
        ssum = jnp.concatenate(
            [a_ref[0] + y2_ref[0], a_ref[1] + y2_ref[1]], axis=-1)
        o_ref[...] = dinv * ssum + b2_ref[...]

    return pl.pallas_call(
        body,
        grid=(N // _BR,),
        in_specs=[
            pl.BlockSpec((2, _BR, C // 2), lambda i: (0, i, 0)),
            pl.BlockSpec((2, _BR, C // 2), lambda i: (0, i, 0)),
            pl.BlockSpec((_BR, 16), lambda i: (i, 0)),
            pl.BlockSpec((_BR, 16), lambda i: (i, 0)),
            pl.BlockSpec((1, C), lambda i: (0, 0)),
        ],
        out_specs=pl.BlockSpec((_BR, C), lambda i: (i, 0)),
        out_shape=jax.ShapeDtypeStruct((N, C), jnp.float32),
    )(a, y2, d0, d1, b2)


def kernel(inputs, edge_index, W1, b1, W2, b2):
    src32 = edge_index[0].astype(jnp.int32)
    dst32 = edge_index[1].astype(jnp.int32)
    src_s = src32.reshape(NS, NBS, K)
    dst_s = dst32.reshape(NS, NBS, K)
    dst_w = dst32.reshape(NW, NBD, K)
    ones16 = jnp.ones((K, 16), jnp.float32)
    zeros16 = jnp.zeros((N, 16), jnp.float32)
    zeros64 = jnp.zeros((N, 64), jnp.float32)
    zeros32 = jnp.zeros((N, 32), jnp.float32)

    xw1 = _tc_matmul(inputs, W1)
    dega = _get_sc_degree()(dst_w, ones16, zeros16)
    d0, d1 = dega[0], dega[1]

    y1 = _tc_scale_split(xw1, d0, d1)
    acc1 = _make_sc_segsum(64)(y1, src_s, dst_s, zeros64)
    y2 = _tc_mid(acc1, y1, d0, d1, b1.reshape(1, -1), W2)
    acc2 = _make_sc_segsum(32)(y2, src_s, dst_s, zeros32)
    return _tc_final(acc2, y2, d0, d1, b2.reshape(1, -1))


# speedup vs baseline: 1.0217x; 1.0217x over previous
import functools

import jax
import jax.numpy as jnp
from jax import lax
from jax.experimental import pallas as pl
from jax.experimental.pallas import tpu as pltpu
from jax.experimental.pallas import tpu_sc as plsc

N = 10000
E = 320000
NC, NS = 2, 16
NW = NC * NS
K = 80
NBD = E // NW // K
NBS = E // NS // K
NBUF = {64: 5, 32: 10}
TPB = N // NS

_SC_PARAMS = dict(compiler_params=pltpu.CompilerParams(use_tc_tiling_on_sc=False))


def _mesh():
    return plsc.VectorSubcoreMesh(core_axis_name="c", subcore_axis_name="s")


@functools.cache
def _get_sc_degree():
    @functools.partial(
        pl.kernel,
        mesh=_mesh(),
        out_type=jax.ShapeDtypeStruct((NC, N, 16), jnp.float32),
        scratch_types=[
            pltpu.VMEM((NBD, K), jnp.int32),
            pltpu.VMEM((K, 16), jnp.float32),
            pltpu.VMEM_SHARED((N, 16), jnp.float32),
        ],
        **_SC_PARAMS,
    )
    def _sc_degree(dst_hbm, ones_hbm, zeros_hbm, out_hbm, dstv, onesv, acc):
        c = lax.axis_index("c")
        s = lax.axis_index("s")
        wid = s * NC + c
        pltpu.sync_copy(zeros_hbm.at[pl.ds(s * TPB, TPB)], acc.at[pl.ds(s * TPB, TPB)])
        pltpu.sync_copy(dst_hbm.at[wid], dstv)
        pltpu.sync_copy(ones_hbm, onesv)
        plsc.subcore_barrier()

        def body(i, carry):
            pltpu.sync_copy(onesv, acc.at[dstv.at[i]], add=True)
            return carry

        lax.fori_loop(0, NBD, body, 0)
        plsc.subcore_barrier()
        pltpu.sync_copy(acc.at[pl.ds(s * TPB, TPB)], out_hbm.at[c, pl.ds(s * TPB, TPB)])

    return _sc_degree


@functools.cache
def _make_sc_segsum(Dh):
    nbuf = NBUF[Dh]

    @functools.partial(
        pl.kernel,
        mesh=_mesh(),
        out_type=jax.ShapeDtypeStruct((NC, N, Dh), jnp.float32),
        scratch_types=[
            pltpu.VMEM((NBS, K), jnp.int32),
            pltpu.VMEM((NBS, K), jnp.int32),
            [pltpu.VMEM((K, Dh), jnp.float32) for _ in range(nbuf)],
            pltpu.VMEM_SHARED((N, Dh), jnp.float32),
            pltpu.SemaphoreType.DMA,
        ],
        **_SC_PARAMS,
    )
    def _sc_segsum(y_hbm, src_hbm, dst_hbm, out_hbm,
                   srcv, dstv, bufs, acc, sem):
        c = lax.axis_index("c")
        s = lax.axis_index("s")
        pltpu.sync_copy(src_hbm.at[s], srcv)
        pltpu.sync_copy(dst_hbm.at[s], dstv)

        def run(cc):
            tbl = y_hbm.at[cc]
            pltpu.sync_copy(tbl.at[pl.ds(s * TPB, TPB)], acc.at[pl.ds(s * TPB, TPB)])
            plsc.subcore_barrier()
            for b in range(nbuf):
                pltpu.async_copy(tbl.at[srcv.at[b]], bufs[b], sem)

            def body(j, carry):
                for b in range(nbuf):
                    i = j * nbuf + b
                    pltpu.make_async_copy(tbl.at[srcv.at[i]], bufs[b], sem).wait()
                    pltpu.sync_copy(bufs[b], acc.at[dstv.at[i]], add=True)

                    @pl.when(i + nbuf < NBS)
                    def _():
                        pltpu.async_copy(tbl.at[srcv.at[i + nbuf]], bufs[b], sem)

                return carry

            lax.fori_loop(0, NBS // nbuf, body, 0)

        @pl.when(c == 0)
        def _():
            run(0)

        @pl.when(c == 1)
        def _():
            run(1)

        plsc.subcore_barrier()
        pltpu.sync_copy(acc.at[pl.ds(s * TPB, TPB)],
                        out_hbm.at[c, pl.ds(s * TPB, TPB)])

    return _sc_segsum


_BR = 1000


def _dinv_block(d0_ref, d1_ref):
    deg = 1.0 + d0_ref[:, 0:1] + d1_ref[:, 0:1]
    return lax.rsqrt(deg)


def _tc_matmul(x, W):
    F, H = W.shape

    def body(x_ref, w_ref, o_ref):
        o_ref[...] = jnp.dot(x_ref[...], w_ref[...],
                             preferred_element_type=jnp.float32)

    return pl.pallas_call(
        body,
        grid=(N // _BR,),
        in_specs=[
            pl.BlockSpec((_BR, F), lambda i: (i, 0)),
            pl.BlockSpec((F, H), lambda i: (0, 0)),
        ],
        out_specs=pl.BlockSpec((_BR, H), lambda i: (i, 0)),
        out_shape=jax.ShapeDtypeStruct((N, H), jnp.float32),
    )(x, W)


def _tc_scale_split(xw, d0, d1):
    H = xw.shape[1]

    def body(xw_ref, d0_ref, d1_ref, y_ref):
        dinv = _dinv_block(d0_ref, d1_ref)
        yblk = xw_ref[...] * dinv
        y_ref[0, :, :] = yblk[:, : H // 2]
        y_ref[1, :, :] = yblk[:, H // 2:]

    return pl.pallas_call(
        body,
        grid=(N // _BR,),
        in_specs=[
            pl.BlockSpec((_BR, H), lambda i: (i, 0)),
            pl.BlockSpec((_BR, 16), lambda i: (i, 0)),
            pl.BlockSpec((_BR, 16), lambda i: (i, 0)),
        ],
        out_specs=pl.BlockSpec((2, _BR, H // 2), lambda i: (0, i, 0)),
        out_shape=jax.ShapeDtypeStruct((2, N, H // 2), jnp.float32),
    )(xw, d0, d1)


def _tc_mid(a, d0, d1, b1, W2):
    H, C = W2.shape

    def body(a_ref, d0_ref, d1_ref, b1_ref, w2_ref, y2_ref):
        dinv = _dinv_block(d0_ref, d1_ref)
        ssum = jnp.concatenate([a_ref[0], a_ref[1]], axis=-1)
        h = jnp.maximum(dinv * ssum + b1_ref[...], 0.0)
        yy = jnp.dot(h, w2_ref[...], preferred_element_type=jnp.float32) * dinv
        y2_ref[0, :, :] = yy[:, : C // 2]
        y2_ref[1, :, :] = yy[:, C // 2:]

    return pl.pallas_call(
        body,
        grid=(N // _BR,),
        in_specs=[
            pl.BlockSpec((2, _BR, H // 2), lambda i: (0, i, 0)),
            pl.BlockSpec((_BR, 16), lambda i: (i, 0)),
            pl.BlockSpec((_BR, 16), lambda i: (i, 0)),
            pl.BlockSpec((1, H), lambda i: (0, 0)),
            pl.BlockSpec((H, C), lambda i: (0, 0)),
        ],
        out_specs=pl.BlockSpec((2, _BR, C // 2), lambda i: (0, i, 0)),
        out_shape=jax.ShapeDtypeStruct((2, N, C // 2), jnp.float32),
    )(a, d0, d1, b1, W2)


def _tc_final(a, d0, d1, b2):
    C = 2 * a.shape[2]

    def body(a_ref, d0_ref, d1_ref, b2_ref, o_ref):
        dinv = _dinv_block(d0_ref, d1_ref)
        ssum = jnp.concatenate([a_ref[0], a_ref[1]], axis=-1)
        o_ref[...] = dinv * ssum + b2_ref[...]

    return pl.pallas_call(
        body,
        grid=(N // _BR,),
        in_specs=[
            pl.BlockSpec((2, _BR, C // 2), lambda i: (0, i, 0)),
            pl.BlockSpec((_BR, 16), lambda i: (i, 0)),
            pl.BlockSpec((_BR, 16), lambda i: (i, 0)),
            pl.BlockSpec((1, C), lambda i: (0, 0)),
        ],
        out_specs=pl.BlockSpec((_BR, C), lambda i: (i, 0)),
        out_shape=jax.ShapeDtypeStruct((N, C), jnp.float32),
    )(a, d0, d1, b2)


def kernel(inputs, edge_index, W1, b1, W2, b2):
    src32 = edge_index[0].astype(jnp.int32)
    dst32 = edge_index[1].astype(jnp.int32)
    src_s = src32.reshape(NS, NBS, K)
    dst_s = dst32.reshape(NS, NBS, K)
    dst_w = dst32.reshape(NW, NBD, K)
    ones16 = jnp.ones((K, 16), jnp.float32)
    zeros16 = jnp.zeros((N, 16), jnp.float32)

    xw1 = _tc_matmul(inputs, W1)
    dega = _get_sc_degree()(dst_w, ones16, zeros16)
    d0, d1 = dega[0], dega[1]

    y1 = _tc_scale_split(xw1, d0, d1)
    acc1 = _make_sc_segsum(64)(y1, src_s, dst_s)
    y2 = _tc_mid(acc1, d0, d1, b1.reshape(1, -1), W2)
    acc2 = _make_sc_segsum(32)(y2, src_s, dst_s)
    return _tc_final(acc2, d0, d1, b2.reshape(1, -1))
